# baseline (speedup 1.0000x reference)
import functools

import jax
import jax.numpy as jnp
from jax import lax
from jax.experimental import pallas as pl
from jax.experimental.pallas import tpu as pltpu
from jax.experimental.pallas import tpu_sc as plsc

B = 16384
SED = 32
PED = 64
NUM_MODES = 12


def _sc_gather(table1d, idx2):
    NW, b_per_w = idx2.shape
    mesh = plsc.VectorSubcoreMesh(core_axis_name="c", subcore_axis_name="s")
    nc = mesh.num_cores

    @functools.partial(
        pl.kernel,
        out_type=jax.ShapeDtypeStruct((NW * b_per_w * SED,), jnp.float32),
        mesh=mesh,
        scratch_types=[
            pltpu.VMEM((b_per_w,), jnp.int32),
            pltpu.VMEM((b_per_w * SED,), jnp.float32),
            pltpu.SemaphoreType.DMA,
        ],
    )
    def gather_kernel(table_hbm, idx_hbm, out_hbm, idx_v, rows_v, sem):
        wid = lax.axis_index("s") * nc + lax.axis_index("c")
        base = pl.multiple_of(wid * (b_per_w * SED), b_per_w * SED)
        pltpu.sync_copy(idx_hbm.at[wid], idx_v)

        def body(g, carry):
            v = idx_v[pl.ds(g * 16, 16)]
            for l in range(16):
                r = pl.multiple_of(v[l] * SED, SED)
                pltpu.async_copy(
                    table_hbm.at[pl.ds(r, SED)],
                    rows_v.at[pl.ds((g * 16 + l) * SED, SED)],
                    sem,
                )
            return carry

        lax.fori_loop(0, b_per_w // 16, body, 0)
        pltpu.make_async_copy(
            table_hbm.at[pl.ds(0, b_per_w * SED)], rows_v, sem
        ).wait()
        pltpu.sync_copy(rows_v, out_hbm.at[pl.ds(base, b_per_w * SED)])

    return gather_kernel(table1d, idx2)


def _tc_fused(rows, tm2d, timestamp, mode_table, W_time, b_time2d, W_pref,
              b_pref2d):
    bs = 2048
    grid = (B // bs,)

    def body(u_ref, tm_ref, ts_ref, mt_ref, wt_ref, bt_ref, wp_ref, bp_ref,
             o_ref):
        u = u_ref[...]
        ts = ts_ref[...]
        tm = tm_ref[...]
        wp = wp_ref[...]
        time_emb = jnp.dot(ts, wt_ref[...], preferred_element_type=jnp.float32)
        time_emb = time_emb + bt_ref[...]
        onehot = (tm == lax.broadcasted_iota(jnp.int32, (bs, NUM_MODES), 1)).astype(
            jnp.float32
        )
        mode_emb = jnp.dot(onehot, mt_ref[...], preferred_element_type=jnp.float32)
        out = jnp.dot(u, wp[0:SED], preferred_element_type=jnp.float32)
        out = out + jnp.dot(mode_emb, wp[SED : 2 * SED], preferred_element_type=jnp.float32)
        out = out + jnp.dot(time_emb, wp[2 * SED :], preferred_element_type=jnp.float32)
        o_ref[...] = out + bp_ref[...]

    return pl.pallas_call(
        body,
        grid=grid,
        in_specs=[
            pl.BlockSpec((bs, SED), lambda i: (i, 0)),
            pl.BlockSpec((bs, 1), lambda i: (i, 0)),
            pl.BlockSpec((bs, 6), lambda i: (i, 0)),
            pl.BlockSpec((NUM_MODES, SED), lambda i: (0, 0)),
            pl.BlockSpec((6, SED), lambda i: (0, 0)),
            pl.BlockSpec((1, SED), lambda i: (0, 0)),
            pl.BlockSpec((3 * SED, PED), lambda i: (0, 0)),
            pl.BlockSpec((1, PED), lambda i: (0, 0)),
        ],
        out_specs=pl.BlockSpec((bs, PED), lambda i: (i, 0)),
        out_shape=jax.ShapeDtypeStruct((B, PED), jnp.float32),
    )(rows, tm2d, timestamp, mode_table, W_time, b_time2d, W_pref, b_pref2d)


def kernel(user_id, transport_mode, timestamp, user_table, mode_table,
           W_time, b_time, W_pref, b_pref):
    info = plsc.get_sparse_core_info()
    NW = info.num_cores * info.num_subcores
    uid = user_id.astype(jnp.int32)
    idx2 = uid.reshape(NW, B // NW)
    rows = _sc_gather(user_table.reshape(-1), idx2).reshape(B, SED)
    return _tc_fused(
        rows,
        transport_mode.astype(jnp.int32).reshape(B, 1),
        timestamp,
        mode_table,
        W_time,
        b_time.reshape(1, SED),
        W_pref,
        b_pref.reshape(1, PED),
    )


# speedup vs baseline: 1.7165x; 1.7165x over previous
import functools

import jax
import jax.numpy as jnp
from jax import lax
from jax.experimental import pallas as pl
from jax.experimental.pallas import tpu as pltpu
from jax.experimental.pallas import tpu_sc as plsc

B = 16384
SED = 32
PED = 64
NUM_MODES = 12


def _sc_gather(user_table, idx2):
    NW, b_per_w = idx2.shape
    mesh = plsc.VectorSubcoreMesh(core_axis_name="c", subcore_axis_name="s")
    nc = mesh.num_cores

    @functools.partial(
        pl.kernel,
        out_type=jax.ShapeDtypeStruct((NW * b_per_w, SED), jnp.float32),
        mesh=mesh,
        scratch_types=[
            pltpu.VMEM((b_per_w,), jnp.int32),
            pltpu.VMEM((b_per_w, SED), jnp.float32),
            pltpu.SemaphoreType.DMA,
        ],
    )
    def gather_kernel(table_hbm, idx_hbm, out_hbm, idx_v, rows_v, sem):
        wid = lax.axis_index("s") * nc + lax.axis_index("c")
        base = wid * b_per_w
        pltpu.sync_copy(idx_hbm.at[wid], idx_v)

        def body(g, carry):
            v = idx_v[pl.ds(g * 16, 16)]
            for l in range(16):
                r = v[l]
                pltpu.async_copy(
                    table_hbm.at[pl.ds(r, 1)],
                    rows_v.at[pl.ds(g * 16 + l, 1)],
                    sem,
                )
            return carry

        lax.fori_loop(0, b_per_w // 16, body, 0)
        pltpu.make_async_copy(
            table_hbm.at[pl.ds(0, b_per_w)], rows_v, sem
        ).wait()
        pltpu.sync_copy(rows_v, out_hbm.at[pl.ds(base, b_per_w)])

    return gather_kernel(user_table, idx2)


def _tc_fused_t(rows, tmT, tsT, mode_table, W_time, b_time2d, WpT, b_pref2d):
    bs = 2048
    grid = (B // bs,)

    def body(u_ref, tm_ref, ts_ref, mt_ref, wt_ref, bt_ref, wpt_ref, bp_ref,
             o_ref):
        u = u_ref[...]
        ts = ts_ref[...]
        tm = tm_ref[...]
        wpt = wpt_ref[...]
        time_embT = lax.dot_general(
            wt_ref[...], ts, (((0,), (0,)), ((), ())),
            preferred_element_type=jnp.float32,
        ) + bt_ref[...]
        onehotT = (
            lax.broadcasted_iota(jnp.int32, (NUM_MODES, bs), 0) == tm
        ).astype(jnp.float32)
        mode_embT = lax.dot_general(
            mt_ref[...], onehotT, (((0,), (0,)), ((), ())),
            preferred_element_type=jnp.float32,
        )
        acc = lax.dot_general(
            wpt[:, 0:SED], u, (((1,), (1,)), ((), ())),
            preferred_element_type=jnp.float32,
        )
        acc = acc + lax.dot_general(
            wpt[:, SED : 2 * SED], mode_embT, (((1,), (0,)), ((), ())),
            preferred_element_type=jnp.float32,
        )
        acc = acc + lax.dot_general(
            wpt[:, 2 * SED :], time_embT, (((1,), (0,)), ((), ())),
            preferred_element_type=jnp.float32,
        )
        o_ref[...] = acc + bp_ref[...]

    return pl.pallas_call(
        body,
        grid=grid,
        in_specs=[
            pl.BlockSpec((bs, SED), lambda i: (i, 0)),
            pl.BlockSpec((1, bs), lambda i: (0, i)),
            pl.BlockSpec((6, bs), lambda i: (0, i)),
            pl.BlockSpec((NUM_MODES, SED), lambda i: (0, 0)),
            pl.BlockSpec((6, SED), lambda i: (0, 0)),
            pl.BlockSpec((SED, 1), lambda i: (0, 0)),
            pl.BlockSpec((PED, 3 * SED), lambda i: (0, 0)),
            pl.BlockSpec((PED, 1), lambda i: (0, 0)),
        ],
        out_specs=pl.BlockSpec((PED, bs), lambda i: (0, i)),
        out_shape=jax.ShapeDtypeStruct((PED, B), jnp.float32),
    )(rows, tmT, tsT, mode_table, W_time, b_time2d, WpT, b_pref2d)


def kernel(user_id, transport_mode, timestamp, user_table, mode_table,
           W_time, b_time, W_pref, b_pref):
    info = plsc.get_sparse_core_info()
    NW = info.num_cores * info.num_subcores
    uid = user_id.astype(jnp.int32)
    idx2 = uid.reshape(NW, B // NW)
    rows = _sc_gather(user_table, idx2)
    outT = _tc_fused_t(
        rows,
        transport_mode.astype(jnp.int32).reshape(1, B),
        timestamp.T,
        mode_table,
        W_time,
        b_time.reshape(SED, 1),
        W_pref.T,
        b_pref.reshape(PED, 1),
    )
    return outT.T
